# baseline (device time: 120478 ns/iter reference)
import jax
import jax.numpy as jnp
from jax import lax
from jax.experimental import pallas as pl
from jax.experimental.pallas import tpu as pltpu

N_DEV = 8
N_EXP = 64
E_LOC = N_EXP // N_DEV
CAP = 25
SLOTS = 32
ROWS = E_LOC * SLOTS
N_TOK = 2048
D = 1024


def _allgather_moe(x_loc, expert_W):

    def body(x_ref, w_ref, out_ref, send_sems, recv_sems):
        my = lax.axis_index("i")
        left = lax.rem(my + N_DEV - 1, N_DEV)
        right = lax.rem(my + 1, N_DEV)

        barrier_sem = pltpu.get_barrier_semaphore()
        for nbr in (left, right):
            pl.semaphore_signal(
                barrier_sem, inc=1,
                device_id=(nbr,), device_id_type=pl.DeviceIdType.MESH,
            )
        pl.semaphore_wait(barrier_sem, 2)

        for e in range(E_LOC):
            y = jnp.dot(
                x_ref[e * SLOTS:(e + 1) * SLOTS, :],
                w_ref[e].astype(jnp.bfloat16),
                preferred_element_type=jnp.float32,
            )
            out_ref[pl.ds(my * ROWS + e * SLOTS, SLOTS), :] = y.astype(
                jnp.bfloat16
            )

        for h in range(N_DEV - 1):
            origin = lax.rem(my + N_DEV - h, N_DEV)
            rdma = pltpu.make_async_remote_copy(
                src_ref=out_ref.at[pl.ds(origin * ROWS, ROWS)],
                dst_ref=out_ref.at[pl.ds(origin * ROWS, ROWS)],
                send_sem=send_sems.at[h],
                recv_sem=recv_sems.at[h],
                device_id=(right,),
                device_id_type=pl.DeviceIdType.MESH,
            )
            rdma.start()
            rdma.wait()

    return pl.pallas_call(
        body,
        out_shape=jax.ShapeDtypeStruct((N_DEV * ROWS, D), jnp.bfloat16),
        in_specs=[
            pl.BlockSpec(memory_space=pltpu.VMEM),
            pl.BlockSpec(memory_space=pltpu.VMEM),
        ],
        out_specs=pl.BlockSpec(memory_space=pltpu.VMEM),
        scratch_shapes=[
            pltpu.SemaphoreType.DMA((N_DEV - 1,)),
            pltpu.SemaphoreType.DMA((N_DEV - 1,)),
        ],
        compiler_params=pltpu.CompilerParams(collective_id=0),
    )(x_loc, expert_W)


def kernel(x, router_W, route_idx, expert_W):
    del router_W
    my = lax.axis_index("i")

    route = route_idx[:, 0]
    onehot = route[:, None] == jnp.arange(N_EXP, dtype=route.dtype)[None, :]
    rank = (
        jnp.take_along_axis(
            jnp.cumsum(onehot.astype(jnp.int32), axis=0),
            route[:, None], axis=1,
        )[:, 0]
        - 1
    )
    keep = rank < CAP

    is_mine = (route // E_LOC) == my
    slot = jnp.where(is_mine & keep, (route % E_LOC) * SLOTS + rank, ROWS)
    x_loc = (
        jnp.zeros((ROWS, D), jnp.bfloat16)
        .at[slot]
        .set(x.astype(jnp.bfloat16), mode="drop")
    )

    y_all = _allgather_moe(x_loc, expert_W)

    src = (route // E_LOC) * ROWS + (route % E_LOC) * SLOTS + rank
    src = jnp.clip(src, 0, N_DEV * ROWS - 1)
    out = jnp.where(keep[:, None], y_all[src], jnp.bfloat16(0))
    return out.astype(jnp.float32)
